# baseline (speedup 1.0000x reference)
import functools

import jax
import jax.numpy as jnp
from jax import lax
from jax.experimental import pallas as pl
from jax.experimental.pallas import tpu as pltpu
from jax.experimental.pallas import tpu_sc as plsc

NC, NS = 2, 16
NW = NC * NS
E = 320000
B = 128
NBLK = E // B
NB0 = NBLK // NW
REM = NBLK - NB0 * NW
D = 128
DE = 16
DO = D + DE + D
TR = DO // 8
TRS = D // 8
TRE = DE // 8
L = 16
GP = D + 1

_mesh = plsc.VectorSubcoreMesh(
    core_axis_name="c", subcore_axis_name="s", num_cores=NC, num_subcores=NS
)


@functools.partial(
    pl.kernel,
    out_type=jax.ShapeDtypeStruct((TR, NBLK, 8, B), jnp.float32),
    mesh=_mesh,
    scratch_types=[
        pltpu.VMEM((B * GP,), jnp.float32),
    ] + [pltpu.VMEM((B,), jnp.int32)] * 2
      + [pltpu.VMEM((B,), jnp.int32)] * 2
      + [pltpu.VMEM((B, D), jnp.float32)] * 2
      + [pltpu.VMEM((TR, 8, B), jnp.float32)] * 2
      + [pltpu.SemaphoreType.DMA] * 10,
    compiler_params=pltpu.CompilerParams(use_tc_tiling_on_sc=True,
                                         needs_layout_passes=False),
)
def _triplet_cat_sc(x_hbm, ett_hbm, src_hbm, dst_hbm, out_hbm,
                    gp, si0, si1, di0, di1, g0, g1, ot0, ot1,
                    gs0, gs1, gd0, gd1, se0, se1, st0, st1, ix0, ix1):
    wid = lax.axis_index("s") * NC + lax.axis_index("c")
    nb = jnp.where(wid < REM, NB0 + 1, NB0)
    sb = NB0 * wid + jnp.minimum(wid, REM)
    end = sb + nb
    sidx = [si0, si1]
    didx = [di0, di1]
    g = [g0, g1]
    otb = [ot0, ot1]
    sem_gs = [gs0, gs1]
    sem_gd = [gd0, gd1]
    sem_e = [se0, se1]
    sem_st = [st0, st1]
    sem_ix = [ix0, ix1]
    iotav = lax.iota(jnp.int32, L)

    def idx_copies(c, s):
        base = c * B
        return (
            pltpu.make_async_copy(src_hbm.at[pl.ds(base, B)], sidx[s], sem_ix[s]),
            pltpu.make_async_copy(dst_hbm.at[pl.ds(base, B)], didx[s], sem_ix[s]),
        )

    def src_gather(s):
        return pltpu.make_async_copy(x_hbm.at[sidx[s]], g[s], sem_gs[s])

    def dst_gather(s):
        return pltpu.make_async_copy(x_hbm.at[didx[s]], g[s], sem_gd[s])

    def edge_copy(c, s):
        return pltpu.make_async_copy(
            ett_hbm.at[:, c], otb[s].at[pl.ds(TRS, TRE)], sem_e[s])

    def out_copy(c, s):
        return pltpu.make_async_copy(otb[s], out_hbm.at[:, c], sem_st[s])

    def pass1(s):
        gref = g[s]

        @plsc.parallel_loop(0, B, unroll=2)
        def p1(e):
            gv = jnp.full((L,), e, jnp.int32)
            for k in range(D // L):
                v = plsc.load_gather(gref, [gv, k * L + iotav])
                gp[pl.ds(e * GP + k * L, L)] = v

    def pass2(s, tr0):
        oref = otb[s]

        @plsc.parallel_loop(0, TRS, unroll=2)
        def p2(t):
            for fr in range(8):
                cv = jnp.full((L,), 8 * t + fr, jnp.int32)
                for e0 in range(B // L):
                    v = plsc.load_gather(gp, [(e0 * L + iotav) * GP + cv])
                    oref[tr0 + t, fr, pl.ds(e0 * L, L)] = v

    def blk(c, s, not_first):
        src_gather(s).wait()
        pass1(s)
        dst_gather(s).start()
        @pl.when(not_first)
        def _():
            out_copy(c, s).wait()
        edge_copy(c, s).start()
        pass2(s, 0)
        dst_gather(s).wait()
        pass1(s)

        @pl.when(c + 2 < end)
        def _():
            for cp in idx_copies(c + 2, s):
                cp.start()
        pass2(s, TRS + TRE)
        edge_copy(c, s).wait()
        out_copy(c, s).start()

        @pl.when(c + 2 < end)
        def _():
            for cp in idx_copies(c + 2, s):
                cp.wait()
            src_gather(s).start()

    for s in range(2):
        for cp in idx_copies(sb + s, s):
            cp.start()
        for cp in idx_copies(sb + s, s):
            cp.wait()
        src_gather(s).start()

    def body(i, carry):
        blk(sb + 2 * i, 0, i > 0)
        blk(sb + 2 * i + 1, 1, i > 0)
        return carry

    lax.fori_loop(0, NB0 // 2, body, 0)

    @pl.when(nb > NB0)
    def _():
        blk(sb + NB0, 0, True)

    for s in range(2):
        out_copy(sb, s).wait()


def kernel(x, edge_emb, edge_index):
    src = edge_index[0].astype(jnp.int32)
    dst = edge_index[1].astype(jnp.int32)
    ett = edge_emb.T.reshape(TRE, 8, NBLK, B).transpose(0, 2, 1, 3)
    ot4 = _triplet_cat_sc(x, ett, src, dst)
    return ot4.transpose(1, 3, 0, 2).reshape(E, DO)


# speedup vs baseline: 1.1287x; 1.1287x over previous
import functools

import jax
import jax.numpy as jnp
from jax import lax
from jax.experimental import pallas as pl
from jax.experimental.pallas import tpu as pltpu
from jax.experimental.pallas import tpu_sc as plsc

NC, NS = 2, 16
NW = NC * NS
E = 320000
B = 128
NBLK = E // B
NB0 = NBLK // NW
REM = NBLK - NB0 * NW
D = 128
DE = 16
DO = D + DE + D
TR = DO // 8
TRS = D // 8
TRE = DE // 8
L = 16
GP = D + 1

_mesh = plsc.VectorSubcoreMesh(
    core_axis_name="c", subcore_axis_name="s", num_cores=NC, num_subcores=NS
)


@functools.partial(
    pl.kernel,
    out_type=jax.ShapeDtypeStruct((TR, NBLK, 8, B), jnp.float32),
    mesh=_mesh,
    scratch_types=[
        pltpu.VMEM((B * GP,), jnp.float32),
    ] + [pltpu.VMEM((B,), jnp.int32)] * 2
      + [pltpu.VMEM((B,), jnp.int32)] * 2
      + [pltpu.VMEM((B, D), jnp.float32)] * 2
      + [pltpu.VMEM((TR, 8, B), jnp.float32)] * 2
      + [pltpu.SemaphoreType.DMA] * 10,
    compiler_params=pltpu.CompilerParams(use_tc_tiling_on_sc=True,
                                         needs_layout_passes=False),
)
def _triplet_cat_sc(x_hbm, ett_hbm, src_hbm, dst_hbm, out_hbm,
                    gp, si0, si1, di0, di1, g0, g1, ot0, ot1,
                    gs0, gs1, gd0, gd1, se0, se1, st0, st1, ix0, ix1):
    wid = lax.axis_index("s") * NC + lax.axis_index("c")
    nb = jnp.where(wid < REM, NB0 + 1, NB0)
    sb = NB0 * wid + jnp.minimum(wid, REM)
    end = sb + nb
    sidx = [si0, si1]
    didx = [di0, di1]
    g = [g0, g1]
    otb = [ot0, ot1]
    sem_gs = [gs0, gs1]
    sem_gd = [gd0, gd1]
    sem_e = [se0, se1]
    sem_st = [st0, st1]
    sem_ix = [ix0, ix1]
    iotav = lax.iota(jnp.int32, L)

    def idx_copies(c, s):
        base = c * B
        return (
            pltpu.make_async_copy(src_hbm.at[pl.ds(base, B)], sidx[s], sem_ix[s]),
            pltpu.make_async_copy(dst_hbm.at[pl.ds(base, B)], didx[s], sem_ix[s]),
        )

    def src_gather(s):
        return pltpu.make_async_copy(x_hbm.at[sidx[s]], g[s], sem_gs[s])

    def dst_gather(s):
        return pltpu.make_async_copy(x_hbm.at[didx[s]], g[s], sem_gd[s])

    def edge_copy(c, s):
        return pltpu.make_async_copy(
            ett_hbm.at[:, c], otb[s].at[pl.ds(TRS, TRE)], sem_e[s])

    def out_copy(c, s):
        return pltpu.make_async_copy(otb[s], out_hbm.at[:, c], sem_st[s])

    def pass1(s):
        gref = g[s]

        @plsc.parallel_loop(0, B, unroll=4)
        def p1(e):
            gv = jnp.full((L,), e, jnp.int32)
            for k in range(D // L):
                v = plsc.load_gather(gref, [gv, k * L + iotav])
                gp[pl.ds(e * GP + k * L, L)] = v

    def pass2(s, tr0):
        oref = otb[s]

        @plsc.parallel_loop(0, TRS, unroll=4)
        def p2(t):
            for fr in range(8):
                cv = jnp.full((L,), 8 * t + fr, jnp.int32)
                for e0 in range(B // L):
                    v = plsc.load_gather(gp, [(e0 * L + iotav) * GP + cv])
                    oref[tr0 + t, fr, pl.ds(e0 * L, L)] = v

    def blk(c, s, not_first):
        src_gather(s).wait()
        pass1(s)
        dst_gather(s).start()
        @pl.when(not_first)
        def _():
            out_copy(c, s).wait()
        edge_copy(c, s).start()
        pass2(s, 0)
        dst_gather(s).wait()
        pass1(s)

        @pl.when(c + 2 < end)
        def _():
            for cp in idx_copies(c + 2, s):
                cp.start()
        pass2(s, TRS + TRE)
        edge_copy(c, s).wait()
        out_copy(c, s).start()

        @pl.when(c + 2 < end)
        def _():
            for cp in idx_copies(c + 2, s):
                cp.wait()
            src_gather(s).start()

    for s in range(2):
        for cp in idx_copies(sb + s, s):
            cp.start()
        for cp in idx_copies(sb + s, s):
            cp.wait()
        src_gather(s).start()

    def body(i, carry):
        blk(sb + 2 * i, 0, i > 0)
        blk(sb + 2 * i + 1, 1, i > 0)
        return carry

    lax.fori_loop(0, NB0 // 2, body, 0)

    @pl.when(nb > NB0)
    def _():
        blk(sb + NB0, 0, True)

    for s in range(2):
        out_copy(sb, s).wait()


def kernel(x, edge_emb, edge_index):
    src = edge_index[0].astype(jnp.int32)
    dst = edge_index[1].astype(jnp.int32)
    ett = edge_emb.T.reshape(TRE, 8, NBLK, B).transpose(0, 2, 1, 3)
    ot4 = _triplet_cat_sc(x, ett, src, dst)
    return ot4.transpose(1, 3, 0, 2).reshape(E, DO)
